# Initial kernel scaffold; baseline (speedup 1.0000x reference)
import jax
import jax.numpy as jnp
from jax.experimental import pallas as pl


def kernel(x, h0, W_ih0, W_hh0, b_ih0, b_hh0, W_ih1, W_hh1, b_ih1, b_hh1, W_int, b_int, W_mix, b_mix):
    raise NotImplementedError("write your pallas kernel here")


# speedup vs baseline: 1.9184x; 1.9184x over previous
import functools

import jax
import jax.numpy as jnp
from jax.experimental import pallas as pl
from jax.experimental.pallas import tpu as pltpu

B, T, D = 64, 512, 128
H = 256
W = 32
N = 64
R = 4
TAU = 0.3
CLIP = 20.0
RW = R * W
CTL_OUT = 2 * RW + W + W
EPS = 1e-8

NB = 2
BH = B // NB
TC = 64
NT = T // TC


def _ebm_kernel(x_ref, h_ref, wx0_ref, wr0_ref, whh0_ref, wih1_ref,
                whh1_ref, wint_ref, wmix_ref, bih0_ref, bhh0_ref,
                bih1_ref, bhh1_ref, bint_ref, bmix_ref, out_ref,
                h0_s, h1_s, em_s, bm_s, lr_s):
    tb = pl.program_id(1)

    @pl.when(tb == 0)
    def _init():
        h0_s[...] = h_ref[0]
        h1_s[...] = h_ref[1]
        em_s[...] = jnp.zeros_like(em_s)
        bm_s[...] = jnp.zeros_like(bm_s)
        lr_s[...] = jnp.zeros_like(lr_s)

    wx0 = wx0_ref[...]
    wr0 = wr0_ref[...]
    whh0 = whh0_ref[...]
    wih1 = wih1_ref[...]
    whh1 = whh1_ref[...]
    wint = wint_ref[...]
    wmix = wmix_ref[...]
    bih0 = bih0_ref[...]
    bhh0 = bhh0_ref[...]
    bih1 = bih1_ref[...]
    bhh1 = bhh1_ref[...]
    bint = bint_ref[...]
    bmix = bmix_ref[...]

    lane_n = jax.lax.broadcasted_iota(jnp.int32, (1, 1, N), 2)

    def _gru(gi, gh, h):
        r = jax.nn.sigmoid(gi[:, :H] + gh[:, :H])
        z = jax.nn.sigmoid(gi[:, H:2 * H] + gh[:, H:2 * H])
        n = jnp.tanh(gi[:, 2 * H:] + r * gh[:, 2 * H:])
        return (1.0 - z) * n + z * h

    def step(tl, dummy):
        t = tb * TC + tl
        xt = x_ref[tl]
        lread = lr_s[...]
        h0c = h0_s[...]
        h1c = h1_s[...]

        gi0 = (jnp.dot(xt, wx0, preferred_element_type=jnp.float32)
               + jnp.dot(lread, wr0, preferred_element_type=jnp.float32)
               + bih0)
        gh0 = jnp.dot(h0c, whh0, preferred_element_type=jnp.float32) + bhh0
        h0n = _gru(gi0, gh0, h0c)

        gi1 = jnp.dot(h0n, wih1, preferred_element_type=jnp.float32) + bih1
        gh1 = jnp.dot(h1c, whh1, preferred_element_type=jnp.float32) + bhh1
        h1n = _gru(gi1, gh1, h1c)

        itf = jnp.dot(h1n, wint, preferred_element_type=jnp.float32) + bint
        itf = jnp.minimum(jnp.maximum(itf, 0.0), CLIP)

        m_t = itf[:, 2 * RW:2 * RW + W]
        e_t = jax.nn.sigmoid(itf[:, 2 * RW + W:])

        slot = t % N
        lmask = lane_n == slot
        em = em_s[...]
        em_er = jnp.sum(jnp.where(lmask, em, 0.0), axis=2)
        em = jnp.where(lmask, m_t[:, :, None], em)
        em_s[...] = em
        bm = bm_s[...]

        em_inv = 1.0 / (jnp.sqrt(jnp.sum(em * em, axis=1, keepdims=True))
                        + EPS)
        bm_inv = 1.0 / (jnp.sqrt(jnp.sum(bm * bm, axis=1, keepdims=True))
                        + EPS)

        def _read(key, mem, mem_inv, inv_tau):
            kn = key * (1.0 / (jnp.sqrt(
                jnp.sum(key * key, axis=-1, keepdims=True)) + EPS))
            s = jnp.sum(kn[:, :, None] * mem, axis=1)
            s = s * mem_inv[:, 0, :] * inv_tau
            p = jnp.exp(s - jnp.max(s, axis=-1, keepdims=True))
            p = p * (1.0 / jnp.sum(p, axis=-1, keepdims=True))
            rd = jnp.sum(p[:, None, :] * mem, axis=2)
            return rd, p

        reads = []
        ww = jnp.zeros((BH, N), jnp.float32)
        for r in range(R):
            k_em = itf[:, W * r:W * (r + 1)]
            k_bm = itf[:, RW + W * r:RW + W * (r + 1)]
            r_em, _ = _read(k_em, em, em_inv, 1.0 / TAU)
            r_bm, p_bm = _read(k_bm, bm, bm_inv, 1.0)
            ww = ww + p_bm
            g = jax.nn.sigmoid(
                jnp.dot(jnp.concatenate([r_em, r_bm], axis=-1), wmix,
                        preferred_element_type=jnp.float32) + bmix)
            reads.append(g * r_em + (1.0 - g) * r_bm)

        ww = ww * (1.0 / R)
        bm = (bm * (1.0 - ww[:, None, :] * e_t[:, :, None])
              + ww[:, None, :] * em_er[:, :, None])
        bm_s[...] = bm

        lnew = jnp.concatenate(reads, axis=-1)
        lr_s[...] = lnew
        h0_s[...] = h0n
        h1_s[...] = h1n

        out_ref[tl] = jnp.concatenate([h0n, h1n, lnew], axis=-1)
        return dummy

    jax.lax.fori_loop(0, TC, step, 0)


@functools.partial(jax.jit, static_argnames=("interpret",))
def _ebm_forward(x, h0, W_ih0, W_hh0, b_ih0, b_hh0, W_ih1, W_hh1, b_ih1,
                 b_hh1, W_int, b_int, W_mix, b_mix, interpret=False):
    xT = jnp.swapaxes(x, 0, 1)
    wx0 = W_ih0[:, :D].T
    wr0 = W_ih0[:, D:].T
    whh0 = W_hh0.T
    wih1 = W_ih1.T
    whh1 = W_hh1.T
    wint = W_int.T
    wmix = W_mix.T
    b2 = lambda b: b.reshape(1, -1)

    full = lambda a: pl.BlockSpec(a.shape, lambda b, t: (0,) * a.ndim)
    outT = pl.pallas_call(
        _ebm_kernel,
        grid=(NB, NT),
        in_specs=[
            pl.BlockSpec((TC, BH, D), lambda b, t: (t, b, 0)),
            pl.BlockSpec((2, BH, H), lambda b, t: (0, b, 0)),
            full(wx0), full(wr0), full(whh0), full(wih1), full(whh1),
            full(wint), full(wmix),
            full(b2(b_ih0)), full(b2(b_hh0)), full(b2(b_ih1)),
            full(b2(b_hh1)), full(b2(b_int)), full(b2(b_mix)),
        ],
        out_specs=pl.BlockSpec((TC, BH, 2 * H + RW), lambda b, t: (t, b, 0)),
        out_shape=jax.ShapeDtypeStruct((T, B, 2 * H + RW), jnp.float32),
        scratch_shapes=[
            pltpu.VMEM((BH, H), jnp.float32),
            pltpu.VMEM((BH, H), jnp.float32),
            pltpu.VMEM((BH, W, N), jnp.float32),
            pltpu.VMEM((BH, W, N), jnp.float32),
            pltpu.VMEM((BH, RW), jnp.float32),
        ],
        compiler_params=pltpu.CompilerParams(
            dimension_semantics=("arbitrary", "arbitrary"),
            vmem_limit_bytes=48 * 1024 * 1024,
        ),
        name="ebm_rnn",
        interpret=interpret,
    )(xT, h0, wx0, wr0, whh0, wih1, whh1, wint, wmix,
      b2(b_ih0), b2(b_hh0), b2(b_ih1), b2(b_hh1), b2(b_int), b2(b_mix))
    return jnp.swapaxes(outT, 0, 1)


def kernel(x, h0, W_ih0, W_hh0, b_ih0, b_hh0, W_ih1, W_hh1, b_ih1, b_hh1,
           W_int, b_int, W_mix, b_mix):
    return _ebm_forward(x, h0, W_ih0, W_hh0, b_ih0, b_hh0, W_ih1, W_hh1,
                        b_ih1, b_hh1, W_int, b_int, W_mix, b_mix)


# speedup vs baseline: 2.4209x; 1.2620x over previous
import functools

import jax
import jax.numpy as jnp
from jax.experimental import pallas as pl
from jax.experimental.pallas import tpu as pltpu

B, T, D = 64, 512, 128
H = 256
W = 32
N = 64
R = 4
TAU = 0.3
CLIP = 20.0
RW = R * W
CTL_OUT = 2 * RW + W + W
EPS = 1e-8

NB = 1
BH = B // NB
TC = 64
NT = T // TC


def _ebm_kernel(x_ref, h_ref, wx0_ref, wr0_ref, whh0_ref, wih1_ref,
                whh1_ref, wint_ref, wmix_ref, bih0_ref, bhh0_ref,
                bih1_ref, bhh1_ref, bint_ref, bmix_ref, out_ref,
                h0_s, h1_s, em_s, bm_s, lr_s, emw_s, emss_s, gi0x_s):
    tb = pl.program_id(1)

    @pl.when(tb == 0)
    def _init():
        h0_s[...] = h_ref[0]
        h1_s[...] = h_ref[1]
        em_s[...] = jnp.zeros_like(em_s)
        bm_s[...] = jnp.zeros_like(bm_s)
        lr_s[...] = jnp.zeros_like(lr_s)
        emw_s[...] = jnp.zeros_like(emw_s)
        emss_s[...] = jnp.zeros_like(emss_s)

    wr0 = wr0_ref[...]
    whh0 = whh0_ref[...]
    wih1 = wih1_ref[...]
    whh1 = whh1_ref[...]
    wint = wint_ref[...]
    wmix = wmix_ref[...]
    bhh0 = bhh0_ref[...]
    bih1 = bih1_ref[...]
    bhh1 = bhh1_ref[...]
    bint = bint_ref[...]
    bmix = bmix_ref[...]

    lane_n = jax.lax.broadcasted_iota(jnp.int32, (1, 1, N), 2)

    gi0x_s[...] = (jnp.dot(x_ref[...].reshape(TC * BH, D), wx0_ref[...],
                           preferred_element_type=jnp.float32)
                   + bih0_ref[...])

    def _gru(gi, gh, h):
        r = jax.nn.sigmoid(gi[:, :H] + gh[:, :H])
        z = jax.nn.sigmoid(gi[:, H:2 * H] + gh[:, H:2 * H])
        n = jnp.tanh(gi[:, 2 * H:] + r * gh[:, 2 * H:])
        return (1.0 - z) * n + z * h

    def step(tl, dummy):
        t = tb * TC + tl
        lread = lr_s[...]
        h0c = h0_s[...]
        h1c = h1_s[...]

        gi0 = (gi0x_s[pl.ds(tl * BH, BH), :]
               + jnp.dot(lread, wr0, preferred_element_type=jnp.float32))
        gh0 = jnp.dot(h0c, whh0, preferred_element_type=jnp.float32) + bhh0
        h0n = _gru(gi0, gh0, h0c)

        gi1 = jnp.dot(h0n, wih1, preferred_element_type=jnp.float32) + bih1
        gh1 = jnp.dot(h1c, whh1, preferred_element_type=jnp.float32) + bhh1
        h1n = _gru(gi1, gh1, h1c)

        itf = jnp.dot(h1n, wint, preferred_element_type=jnp.float32) + bint
        itf = jax.lax.clamp(0.0, itf, CLIP)

        m_t = itf[:, 2 * RW:2 * RW + W]
        e_t = jax.nn.sigmoid(itf[:, 2 * RW + W:])

        slot = t % N
        lmask = lane_n == slot
        em_er = emw_s[slot]
        emw_s[slot] = m_t
        em_s[...] = jnp.where(lmask, m_t[:, :, None], em_s[...])

        m_ss = jnp.sum(m_t * m_t, axis=-1, keepdims=True)
        em_ss = jnp.where(lmask[:, 0, :], m_ss, emss_s[...])
        emss_s[...] = em_ss
        em_inv = 1.0 / (jnp.sqrt(em_ss) + EPS)
        bm = bm_s[...]
        bm_inv = 1.0 / (jnp.sqrt(jnp.sum(bm * bm, axis=1, keepdims=True))
                        + EPS)[:, 0, :]

        def _read(key, mem_ref, mem_inv, inv_tau):
            kn = key * (1.0 / (jnp.sqrt(
                jnp.sum(key * key, axis=-1, keepdims=True)) + EPS))
            s = jnp.sum(kn[:, :, None] * mem_ref[...], axis=1)
            s = s * mem_inv * inv_tau
            p = jnp.exp(s - jnp.max(s, axis=-1, keepdims=True))
            p = p * (1.0 / jnp.sum(p, axis=-1, keepdims=True))
            p8 = jnp.broadcast_to(p[:, None, :], (BH, 8, N))
            p32 = pltpu.repeat(p8, W // 8, axis=1)
            rd = jnp.sum(p32 * mem_ref[...], axis=2)
            return rd, p

        reads = []
        ww = jnp.zeros((BH, N), jnp.float32)
        for r in range(R):
            k_em = itf[:, W * r:W * (r + 1)]
            k_bm = itf[:, RW + W * r:RW + W * (r + 1)]
            r_em, _ = _read(k_em, em_s, em_inv, 1.0 / TAU)
            r_bm, p_bm = _read(k_bm, bm_s, bm_inv, 1.0)
            ww = ww + p_bm
            g = jax.nn.sigmoid(
                jnp.dot(jnp.concatenate([r_em, r_bm], axis=-1), wmix,
                        preferred_element_type=jnp.float32) + bmix)
            reads.append(g * r_em + (1.0 - g) * r_bm)

        ww = ww * (1.0 / R)
        scale = 1.0 - ww[:, None, :] * e_t[:, :, None]
        bm_s[...] = bm_s[...] * scale + ww[:, None, :] * em_er[:, :, None]

        lnew = jnp.concatenate(reads, axis=-1)
        lr_s[...] = lnew
        h0_s[...] = h0n
        h1_s[...] = h1n

        out_ref[tl] = jnp.concatenate([h0n, h1n, lnew], axis=-1)
        return dummy

    jax.lax.fori_loop(0, TC, step, 0, unroll=4)


@functools.partial(jax.jit, static_argnames=("interpret",))
def _ebm_forward(x, h0, W_ih0, W_hh0, b_ih0, b_hh0, W_ih1, W_hh1, b_ih1,
                 b_hh1, W_int, b_int, W_mix, b_mix, interpret=False):
    xT = jnp.swapaxes(x, 0, 1)
    wx0 = W_ih0[:, :D].T
    wr0 = W_ih0[:, D:].T
    whh0 = W_hh0.T
    wih1 = W_ih1.T
    whh1 = W_hh1.T
    wint = W_int.T
    wmix = W_mix.T
    b2 = lambda b: b.reshape(1, -1)

    full = lambda a: pl.BlockSpec(a.shape, lambda b, t: (0,) * a.ndim)
    outT = pl.pallas_call(
        _ebm_kernel,
        grid=(NB, NT),
        in_specs=[
            pl.BlockSpec((TC, BH, D), lambda b, t: (t, b, 0)),
            pl.BlockSpec((2, BH, H), lambda b, t: (0, b, 0)),
            full(wx0), full(wr0), full(whh0), full(wih1), full(whh1),
            full(wint), full(wmix),
            full(b2(b_ih0)), full(b2(b_hh0)), full(b2(b_ih1)),
            full(b2(b_hh1)), full(b2(b_int)), full(b2(b_mix)),
        ],
        out_specs=pl.BlockSpec((TC, BH, 2 * H + RW), lambda b, t: (t, b, 0)),
        out_shape=jax.ShapeDtypeStruct((T, B, 2 * H + RW), jnp.float32),
        scratch_shapes=[
            pltpu.VMEM((BH, H), jnp.float32),
            pltpu.VMEM((BH, H), jnp.float32),
            pltpu.VMEM((BH, W, N), jnp.float32),
            pltpu.VMEM((BH, W, N), jnp.float32),
            pltpu.VMEM((BH, RW), jnp.float32),
            pltpu.VMEM((N, BH, W), jnp.float32),
            pltpu.VMEM((BH, N), jnp.float32),
            pltpu.VMEM((TC * BH, 3 * H), jnp.float32),
        ],
        compiler_params=pltpu.CompilerParams(
            dimension_semantics=("arbitrary", "arbitrary"),
            vmem_limit_bytes=48 * 1024 * 1024,
        ),
        name="ebm_rnn",
        interpret=interpret,
    )(xT, h0, wx0, wr0, whh0, wih1, whh1, wint, wmix,
      b2(b_ih0), b2(b_hh0), b2(b_ih1), b2(b_hh1), b2(b_int), b2(b_mix))
    return jnp.swapaxes(outT, 0, 1)


def kernel(x, h0, W_ih0, W_hh0, b_ih0, b_hh0, W_ih1, W_hh1, b_ih1, b_hh1,
           W_int, b_int, W_mix, b_mix):
    return _ebm_forward(x, h0, W_ih0, W_hh0, b_ih0, b_hh0, W_ih1, W_hh1,
                        b_ih1, b_hh1, W_int, b_int, W_mix, b_mix)
